# baseline (speedup 1.0000x reference)
import functools

import jax
import jax.numpy as jnp
from jax import lax
from jax.experimental import pallas as pl
from jax.experimental.pallas import tpu as pltpu
from jax.experimental.pallas import tpu_sc as plsc

B = 64
H = 224
W = 224
C5 = 5
C = 8
WC5 = W * C5
PIX = H * W
N = B * PIX
TW5 = N * C5


def _tc_body(x_ref, tab_ref):
    blk = x_ref[0]
    v = blk[0:222, :] + blk[2:224, :]
    t = (v[:, 0 : WC5 - 2 * C5] + v[:, 2 * C5 : WC5]) * 0.25
    tab_ref[0] = jnp.pad(t, ((1, 1), (C5, C5)))


def _tc_stage(x5):
    return pl.pallas_call(
        _tc_body,
        grid=(B,),
        in_specs=[pl.BlockSpec((1, H, WC5), lambda b: (b, 0, 0))],
        out_specs=[pl.BlockSpec((1, H, WC5), lambda b: (b, 0, 0))],
        out_shape=[jax.ShapeDtypeStruct((B, H, WC5), jnp.float32)],
    )(x5)


_NC = 2
_NS = 16
_NW = _NC * _NS
SUB = 128
G = 16
NROW = N // SUB
PER_W = NROW // _NW
N_CHUNKS = PER_W // G
CPX = G * SUB
ECH = CPX * 3

_SC_PARAMS = pltpu.CompilerParams(
    use_tc_tiling_on_sc=False, needs_layout_passes=False
)

_INV224 = 1.0 / 224.0


def _sc_expand(x5f, tab5):
    mesh = plsc.VectorSubcoreMesh(core_axis_name="c", subcore_axis_name="s")
    in_ch = CPX * C5
    out_ch = CPX * C

    @functools.partial(
        pl.kernel,
        mesh=mesh,
        out_type=[
            jax.ShapeDtypeStruct((N * C,), jnp.float32),
            jax.ShapeDtypeStruct((N,), jnp.int32),
        ],
        compiler_params=_SC_PARAMS,
        scratch_types=[
            pltpu.VMEM((in_ch,), jnp.float32),
            pltpu.VMEM((in_ch,), jnp.float32),
            pltpu.VMEM((out_ch,), jnp.float32),
            pltpu.VMEM((CPX,), jnp.int32),
        ],
    )
    def k(x_hbm, tab5_hbm, tab8_hbm, idx_hbm, tin, xin, xout, idxo):
        wid = lax.axis_index("s") * _NC + lax.axis_index("c")
        base_px = wid * (N // _NW)

        lane = lax.iota(jnp.int32, 16)
        pats = []
        for s in range(5):
            q = 16 * s + lane
            q5 = (q * 52429) >> 18
            pats.append(q + 3 * q5)
        dxpat = 5 * lane + 3
        dypat = 5 * lane + 4

        def body(i, _):
            px0 = base_px + i * CPX
            pltpu.sync_copy(tab5_hbm.at[pl.ds(px0 * C5, in_ch)], tin)
            pltpu.sync_copy(x_hbm.at[pl.ds(px0 * C5, in_ch)], xin)

            def group(u, _):
                for s in range(5):
                    vals = tin[pl.ds(80 * u + 16 * s, 16)]
                    plsc.store_scatter(xout, [128 * u + pats[s]], vals)

                dxv = plsc.load_gather(xin, [80 * u + dxpat])
                dyv = plsc.load_gather(xin, [80 * u + dypat])
                px = px0 + 16 * u + lane
                pxf = px.astype(jnp.float32)
                row = ((pxf + 0.5) * _INV224).astype(jnp.int32)
                xc = px - 224 * row
                bb = ((row.astype(jnp.float32) + 0.5) * _INV224).astype(
                    jnp.int32
                )
                yc = row - 224 * bb
                xi = jnp.mod(xc.astype(jnp.float32) + dxv, 224.0).astype(
                    jnp.int32
                )
                yi = jnp.mod(yc.astype(jnp.float32) + dyv, 224.0).astype(
                    jnp.int32
                )
                xi = jnp.minimum(xi, 223)
                yi = jnp.minimum(yi, 223)
                idxo[pl.ds(16 * u, 16)] = (bb * H + yi) * W + xi
                return 0

            lax.fori_loop(0, CPX // 16, group, 0)
            pltpu.sync_copy(xout, tab8_hbm.at[pl.ds(px0 * C, out_ch)])
            pltpu.sync_copy(idxo, idx_hbm.at[pl.ds(px0, CPX)])
            return 0

        lax.fori_loop(0, N_CHUNKS, body, 0)

    return k(x5f, tab5)


def _sc_gather(table, idx2):
    mesh = plsc.VectorSubcoreMesh(core_axis_name="c", subcore_axis_name="s")

    @functools.partial(
        pl.kernel,
        mesh=mesh,
        out_type=jax.ShapeDtypeStruct((N * 3,), jnp.float32),
        compiler_params=_SC_PARAMS,
        scratch_types=[
            pltpu.VMEM((G, SUB), jnp.int32),
            pltpu.VMEM((CPX, C), jnp.float32),
            pltpu.VMEM((ECH,), jnp.float32),
            pltpu.SemaphoreType.DMA,
        ],
    )
    def k(table_hbm, idx_hbm, out_hbm, idx_v, rows_v, cmp_v, sem):
        wid = lax.axis_index("s") * _NC + lax.axis_index("c")
        base = wid * PER_W

        lane = lax.iota(jnp.int32, 16)
        rpat = [((16 * j + lane) * 21846) >> 16 for j in range(3)]
        cpat = [(16 * j + lane) - 3 * rpat[j] for j in range(3)]

        def body(i, _):
            off = base + i * G
            pltpu.sync_copy(idx_hbm.at[pl.ds(off, G)], idx_v)
            descs = [
                pltpu.async_copy(
                    table_hbm.at[idx_v.at[j]],
                    rows_v.at[pl.ds(j * SUB, SUB)],
                    sem,
                )
                for j in range(G)
            ]
            for d in descs:
                d.wait()

            def compact(g, _):
                for j in range(3):
                    vals = plsc.load_gather(rows_v, [16 * g + rpat[j], cpat[j]])
                    cmp_v[pl.ds(48 * g + 16 * j, 16)] = vals
                return 0

            lax.fori_loop(0, CPX // 16, compact, 0)
            pltpu.sync_copy(cmp_v, out_hbm.at[pl.ds(off * SUB * 3, ECH)])
            return 0

        lax.fori_loop(0, N_CHUNKS, body, 0)

    return k(table, idx2)


def kernel(x):
    x5 = x.reshape(B, H, WC5)
    (tab5,) = _tc_stage(x5)
    tab8, idx = _sc_expand(x5.reshape(TW5), tab5.reshape(TW5))
    out = _sc_gather(tab8.reshape(N, C), idx.reshape(NROW, SUB))
    return out.reshape(B, H, W, 3)


# speedup vs baseline: 1.8944x; 1.8944x over previous
import functools

import jax
import jax.numpy as jnp
from jax import lax
from jax.experimental import pallas as pl
from jax.experimental.pallas import tpu as pltpu
from jax.experimental.pallas import tpu_sc as plsc

B = 64
H = 224
W = 224
C = 8
WC = W * C
PIX = H * W
N = B * PIX


def _tc_body(x_ref, dx_ref, dy_ref, tab_ref, idx_ref):
    b = pl.program_id(0)
    blk = x_ref[0]
    v = blk[0:222, :] + blk[2:224, :]
    t = (v[:, 0 : WC - 2 * C] + v[:, 2 * C : WC]) * 0.25
    tab_ref[0] = jnp.pad(t, ((1, 1), (C, C)))

    dx = dx_ref[0]
    dy = dy_ref[0]
    xg = lax.broadcasted_iota(jnp.int32, (H, W), 1).astype(jnp.float32)
    yg = lax.broadcasted_iota(jnp.int32, (H, W), 0).astype(jnp.float32)
    xi = jnp.mod(xg + dx, 224.0).astype(jnp.int32)
    yi = jnp.mod(yg + dy, 224.0).astype(jnp.int32)
    xi = jnp.minimum(xi, 223)
    yi = jnp.minimum(yi, 223)
    idx_ref[0] = (b * H + yi) * W + xi


def _tc_stage(x8, dx, dy):
    return pl.pallas_call(
        _tc_body,
        grid=(B,),
        in_specs=[
            pl.BlockSpec((1, H, WC), lambda b: (b, 0, 0)),
            pl.BlockSpec((1, H, W), lambda b: (b, 0, 0)),
            pl.BlockSpec((1, H, W), lambda b: (b, 0, 0)),
        ],
        out_specs=[
            pl.BlockSpec((1, H, WC), lambda b: (b, 0, 0)),
            pl.BlockSpec((1, H, W), lambda b: (b, 0, 0)),
        ],
        out_shape=[
            jax.ShapeDtypeStruct((B, H, WC), jnp.float32),
            jax.ShapeDtypeStruct((B, H, W), jnp.int32),
        ],
    )(x8, dx, dy)


_NC = 2
_NS = 16
_NW = _NC * _NS
SUB = 128
G = 16
NROW = N // SUB
PER_W = NROW // _NW
N_CHUNKS = PER_W // G


CPX = G * SUB
ECH = CPX * 3


def _sc_gather(table, idx2):
    mesh = plsc.VectorSubcoreMesh(core_axis_name="c", subcore_axis_name="s")

    @functools.partial(
        pl.kernel,
        mesh=mesh,
        out_type=jax.ShapeDtypeStruct((NROW, SUB, C), jnp.float32),
        compiler_params=pltpu.CompilerParams(
            use_tc_tiling_on_sc=False, needs_layout_passes=False
        ),
        scratch_types=[
            pltpu.VMEM((G, SUB), jnp.int32),
            pltpu.VMEM((G, SUB, C), jnp.float32),
            pltpu.SemaphoreType.DMA,
        ],
    )
    def k(table_hbm, idx_hbm, out_hbm, idx_v, rows_v, sem):
        wid = lax.axis_index("s") * _NC + lax.axis_index("c")
        base = wid * PER_W

        def body(i, _):
            off = base + i * G
            pltpu.sync_copy(idx_hbm.at[pl.ds(off, G)], idx_v)
            descs = [
                pltpu.async_copy(
                    table_hbm.at[idx_v.at[j]],
                    rows_v.at[j],
                    sem,
                )
                for j in range(G)
            ]
            for d in descs:
                d.wait()

            pltpu.sync_copy(rows_v, out_hbm.at[pl.ds(off, G)])
            return 0

        lax.fori_loop(0, N_CHUNKS, body, 0)

    return k(table, idx2)


def kernel(x):
    x8 = jnp.pad(x, ((0, 0), (0, 0), (0, 0), (0, 3))).reshape(B, H, WC)
    dx = x[:, :, :, 3]
    dy = x[:, :, :, 4]
    tab, idx = _tc_stage(x8, dx, dy)
    out8 = _sc_gather(tab.reshape(N, C), idx.reshape(NROW, SUB))
    return out8.reshape(B, H, W, C)[..., 0:3]
